# Initial kernel scaffold; baseline (speedup 1.0000x reference)
import jax
import jax.numpy as jnp
from jax.experimental import pallas as pl


def kernel(x, edge_index, edge_type, W1, root1, b1, W2, root2, b2):
    raise NotImplementedError("write your pallas kernel here")


# speedup vs baseline: 3.3939x; 3.3939x over previous
import functools

import jax
import jax.numpy as jnp
from jax import lax
from jax.experimental import pallas as pl
from jax.experimental.pallas import tpu as pltpu
from jax.experimental.pallas import tpu_sc as plsc

N = 10000
R = 8
E = 320000
D = 128
SEGN = N * R

NC = 2
NS = 16
NW = NC * NS
EPW = E // NW
B = 80
NCHUNK = EPW // B
RPS = N // NS
ZR = 125
SEGPS = SEGN // NS
SEGZ = 1000
LW = 16

_MESH = plsc.VectorSubcoreMesh(core_axis_name="c", subcore_axis_name="s")
_SC_PARAMS = pltpu.CompilerParams(use_tc_tiling_on_sc=False)


def _zero_vmem_2d(ref, nrows, ncols):
    def row(i, carry):
        for j in range(ncols // LW):
            ref[i, pl.ds(j * LW, LW)] = jnp.zeros((LW,), jnp.float32)
        return carry
    lax.fori_loop(0, nrows, row, 0)


@functools.partial(
    pl.kernel,
    out_type=jax.ShapeDtypeStruct((NC, SEGN, LW), jnp.float32),
    mesh=_MESH,
    scratch_types=[
        pltpu.VMEM((B,), jnp.int32),
        pltpu.VMEM((B,), jnp.int32),
        pltpu.VMEM((B,), jnp.int32),
        pltpu.VMEM((B, LW), jnp.float32),
        pltpu.VMEM((SEGZ, LW), jnp.float32),
        pltpu.VMEM_SHARED((SEGN, LW), jnp.float32),
    ],
    compiler_params=_SC_PARAMS,
)
def _deg_kernel(dst_hbm, rel_hbm, out_hbm, dstv, relv, segv, onesv, zv, acc):
    cid = lax.axis_index("c")
    sid = lax.axis_index("s")
    wid = cid * NS + sid

    _zero_vmem_2d(zv, SEGZ, LW)

    def orow(i, carry):
        onesv[i, pl.ds(0, LW)] = jnp.ones((LW,), jnp.float32)
        return carry
    lax.fori_loop(0, B, orow, 0)

    for k in range(SEGPS // SEGZ):
        pltpu.sync_copy(zv, acc.at[pl.ds(sid * SEGPS + k * SEGZ, SEGZ)])
    plsc.subcore_barrier()

    base = wid * EPW

    def chunk(c, carry):
        off = base + c * B
        pltpu.sync_copy(dst_hbm.at[pl.ds(off, B)], dstv)
        pltpu.sync_copy(rel_hbm.at[pl.ds(off, B)], relv)
        for j in range(B // LW):
            sl = pl.ds(j * LW, LW)
            segv[sl] = dstv[sl] * R + relv[sl]
        pltpu.sync_copy(onesv, acc.at[segv], add=True)
        return carry
    lax.fori_loop(0, NCHUNK, chunk, 0)

    plsc.subcore_barrier()
    for k in range(SEGPS // SEGZ):
        s = sid * SEGPS + k * SEGZ
        pltpu.sync_copy(acc.at[pl.ds(s, SEGZ)], out_hbm.at[cid, pl.ds(s, SEGZ)])


@functools.partial(
    pl.kernel,
    out_type=jax.ShapeDtypeStruct((NC, N, D), jnp.float32),
    mesh=_MESH,
    scratch_types=[
        pltpu.VMEM((B,), jnp.int32),
        pltpu.VMEM((B,), jnp.int32),
        pltpu.VMEM((B,), jnp.int32),
        pltpu.VMEM((B,), jnp.int32),
        pltpu.VMEM((B,), jnp.int32),
        pltpu.VMEM((B, D), jnp.float32),
        pltpu.VMEM((B, LW), jnp.float32),
        pltpu.VMEM((ZR, D), jnp.float32),
        pltpu.VMEM_SHARED((N, D), jnp.float32),
        pltpu.SemaphoreType.DMA,
        pltpu.SemaphoreType.DMA,
    ],
    compiler_params=_SC_PARAMS,
)
def _agg_kernel(y_hbm, recip_hbm, src_hbm, dst_hbm, rel_hbm, out_hbm,
                srcv, dstv, relv, gv, sv, rows, scl, zv, acc, sem1, sem2):
    cid = lax.axis_index("c")
    sid = lax.axis_index("s")
    wid = cid * NS + sid

    _zero_vmem_2d(zv, ZR, D)
    for k in range(RPS // ZR):
        pltpu.sync_copy(zv, acc.at[pl.ds(sid * RPS + k * ZR, ZR)])
    plsc.subcore_barrier()

    base = wid * EPW

    def chunk(c, carry):
        off = base + c * B
        pltpu.sync_copy(src_hbm.at[pl.ds(off, B)], srcv)
        pltpu.sync_copy(dst_hbm.at[pl.ds(off, B)], dstv)
        pltpu.sync_copy(rel_hbm.at[pl.ds(off, B)], relv)
        for j in range(B // LW):
            sl = pl.ds(j * LW, LW)
            gv[sl] = relv[sl] * N + srcv[sl]
            sv[sl] = dstv[sl] * R + relv[sl]
        cp1 = pltpu.async_copy(y_hbm.at[gv], rows, sem1)
        cp2 = pltpu.async_copy(recip_hbm.at[sv], scl, sem2)
        cp1.wait()
        cp2.wait()

        def edge(e, carry2):
            svec = scl[e]
            for j in range(D // LW):
                sl = pl.ds(j * LW, LW)
                rows[e, sl] = rows[e, sl] * svec
            return carry2
        lax.fori_loop(0, B, edge, 0)

        pltpu.sync_copy(rows, acc.at[dstv], add=True)
        return carry
    lax.fori_loop(0, NCHUNK, chunk, 0)

    plsc.subcore_barrier()
    for k in range(RPS // ZR):
        s = sid * RPS + k * ZR
        pltpu.sync_copy(acc.at[pl.ds(s, ZR)], out_hbm.at[cid, pl.ds(s, ZR)])


_BN = 2000


def _recip_body(degp_ref, recip_ref):
    d = degp_ref[0] + degp_ref[1]
    recip_ref[...] = 1.0 / jnp.maximum(d, 1.0)


def _recip_tc(deg_parts):
    nb = 8
    blk = SEGN // nb
    return pl.pallas_call(
        _recip_body,
        grid=(nb,),
        in_specs=[pl.BlockSpec((NC, blk, LW), lambda i: (0, i, 0))],
        out_specs=pl.BlockSpec((blk, LW), lambda i: (i, 0)),
        out_shape=jax.ShapeDtypeStruct((SEGN, LW), jnp.float32),
    )(deg_parts)


def _transform_body(x_ref, w_ref, y_ref):
    y_ref[...] = jnp.dot(x_ref[...], w_ref[0], preferred_element_type=jnp.float32)


def _transform_tc(x, W):
    nbx = N // _BN
    return pl.pallas_call(
        _transform_body,
        grid=(R, nbx),
        in_specs=[
            pl.BlockSpec((_BN, D), lambda r, i: (i, 0)),
            pl.BlockSpec((1, D, D), lambda r, i: (r, 0, 0)),
        ],
        out_specs=pl.BlockSpec((_BN, D), lambda r, i: (r * nbx + i, 0)),
        out_shape=jax.ShapeDtypeStruct((R * N, D), jnp.float32),
    )(x, W)


def _combine_body(relu, parts_ref, x_ref, root_ref, b_ref, out_ref):
    acc = parts_ref[0] + parts_ref[1]
    acc = acc + jnp.dot(x_ref[...], root_ref[...], preferred_element_type=jnp.float32)
    acc = acc + b_ref[...]
    if relu:
        acc = jnp.maximum(acc, 0.0)
    out_ref[...] = acc


def _combine_tc(parts, x, root, b, relu):
    nbx = N // _BN
    return pl.pallas_call(
        functools.partial(_combine_body, relu),
        grid=(nbx,),
        in_specs=[
            pl.BlockSpec((NC, _BN, D), lambda i: (0, i, 0)),
            pl.BlockSpec((_BN, D), lambda i: (i, 0)),
            pl.BlockSpec((D, D), lambda i: (0, 0)),
            pl.BlockSpec((1, D), lambda i: (0, 0)),
        ],
        out_specs=pl.BlockSpec((_BN, D), lambda i: (i, 0)),
        out_shape=jax.ShapeDtypeStruct((N, D), jnp.float32),
    )(parts, x, root, b)


def _layer(x, W, root, b, src, dst, rel, recip, relu):
    y = _transform_tc(x, W)
    parts = _agg_kernel(y, recip, src, dst, rel)
    return _combine_tc(parts, x, root, b.reshape(1, D), relu)


def kernel(x, edge_index, edge_type, W1, root1, b1, W2, root2, b2):
    src = edge_index[0].astype(jnp.int32)
    dst = edge_index[1].astype(jnp.int32)
    rel = edge_type.astype(jnp.int32)
    deg_parts = _deg_kernel(dst, rel)
    recip = _recip_tc(deg_parts)
    h = _layer(x, W1, root1, b1, src, dst, rel, recip, relu=True)
    return _layer(h, W2, root2, b2, src, dst, rel, recip, relu=False)


# speedup vs baseline: 7.9038x; 2.3288x over previous
import functools

import jax
import jax.numpy as jnp
from jax import lax
from jax.experimental import pallas as pl
from jax.experimental.pallas import tpu as pltpu
from jax.experimental.pallas import tpu_sc as plsc

N = 10000
R = 8
E = 320000
D = 128
SEGN = N * R

NC = 2
NS = 16
NW = NC * NS
EPW = E // NW
B = 80
NCHUNK = EPW // B
RPS = N // NS
ZR = 125
SEGPS = SEGN // NS
SEGZ = 1000
LW = 16

_MESH = plsc.VectorSubcoreMesh(core_axis_name="c", subcore_axis_name="s")
_SC_PARAMS = pltpu.CompilerParams(use_tc_tiling_on_sc=False)


def _zero_vmem_2d(ref, nrows, ncols):
    def row(i, carry):
        for j in range(ncols // LW):
            ref[i, pl.ds(j * LW, LW)] = jnp.zeros((LW,), jnp.float32)
        return carry
    lax.fori_loop(0, nrows, row, 0)


_DEG_LAG = 4


@functools.partial(
    pl.kernel,
    out_type=jax.ShapeDtypeStruct((NC, SEGN, LW), jnp.float32),
    mesh=_MESH,
    scratch_types=[
        pltpu.VMEM((EPW,), jnp.int32),
        pltpu.VMEM((EPW,), jnp.int32),
        pltpu.VMEM((NCHUNK, B), jnp.int32),
        pltpu.VMEM((B, LW), jnp.float32),
        pltpu.VMEM((SEGZ, LW), jnp.float32),
        pltpu.VMEM_SHARED((SEGN, LW), jnp.float32),
        pltpu.SemaphoreType.DMA,
    ],
    compiler_params=_SC_PARAMS,
)
def _deg_kernel(dst_hbm, rel_hbm, out_hbm, dstw, relw, segw, onesv, zv, acc, sem):
    cid = lax.axis_index("c")
    sid = lax.axis_index("s")
    wid = cid * NS + sid
    base = wid * EPW

    pltpu.sync_copy(dst_hbm.at[pl.ds(base, EPW)], dstw)
    pltpu.sync_copy(rel_hbm.at[pl.ds(base, EPW)], relw)

    def seg_chunk(c, carry):
        off = c * B
        for j in range(B // LW):
            sl = pl.ds(j * LW, LW)
            s2 = pl.ds(off + j * LW, LW)
            segw[c, sl] = dstw[s2] * R + relw[s2]
        return carry
    lax.fori_loop(0, NCHUNK, seg_chunk, 0)

    def orow(i, carry):
        onesv[i, pl.ds(0, LW)] = jnp.ones((LW,), jnp.float32)
        return carry
    lax.fori_loop(0, B, orow, 0)

    _zero_vmem_2d(zv, SEGZ, LW)
    for k in range(SEGPS // SEGZ):
        pltpu.sync_copy(zv, acc.at[pl.ds(sid * SEGPS + k * SEGZ, SEGZ)])
    plsc.subcore_barrier()

    def chunk(c, carry):
        pltpu.async_copy(onesv, acc.at[segw.at[c]], sem, add=True)

        @pl.when(c >= _DEG_LAG)
        def _():
            pltpu.make_async_copy(onesv, acc.at[segw.at[0]], sem).wait()
        return carry
    lax.fori_loop(0, NCHUNK, chunk, 0)
    for _ in range(_DEG_LAG):
        pltpu.make_async_copy(onesv, acc.at[segw.at[0]], sem).wait()

    plsc.subcore_barrier()
    for k in range(SEGPS // SEGZ):
        s = sid * SEGPS + k * SEGZ
        pltpu.sync_copy(acc.at[pl.ds(s, SEGZ)], out_hbm.at[cid, pl.ds(s, SEGZ)])


_ZVA = 25


@functools.partial(
    pl.kernel,
    out_type=jax.ShapeDtypeStruct((NC, N, D), jnp.float32),
    mesh=_MESH,
    scratch_types=[
        [pltpu.VMEM((B,), jnp.int32)] * 2,
        [pltpu.VMEM((B,), jnp.int32)] * 2,
        [pltpu.VMEM((B,), jnp.int32)] * 2,
        [pltpu.VMEM((B,), jnp.int32)] * 2,
        [pltpu.VMEM((B,), jnp.int32)] * 2,
        [pltpu.VMEM((B,), jnp.int32)] * 4,
        [pltpu.VMEM((B, D), jnp.float32)] * 2,
        [pltpu.VMEM((B, D), jnp.float32)] * 2,
        [pltpu.VMEM((B, LW), jnp.float32)] * 2,
        pltpu.VMEM((_ZVA, D), jnp.float32),
        pltpu.VMEM_SHARED((N, D), jnp.float32),
        [pltpu.SemaphoreType.DMA] * 2,
        [pltpu.SemaphoreType.DMA] * 2,
        [pltpu.SemaphoreType.DMA] * 2,
        [pltpu.SemaphoreType.DMA] * 2,
    ],
    compiler_params=_SC_PARAMS,
)
def _agg_kernel(y_hbm, recip_hbm, src_hbm, dst_hbm, rel_hbm, out_hbm,
                srcv, dstv, relv, gv, sv, dv, rows, sbuf, scl, zv, acc,
                semi, semg, sems, semw):
    cid = lax.axis_index("c")
    sid = lax.axis_index("s")
    wid = cid * NS + sid
    base = wid * EPW

    def issue_idx(c, p):
        off = base + c * B
        pltpu.async_copy(src_hbm.at[pl.ds(off, B)], srcv[p], semi[p])
        pltpu.async_copy(dst_hbm.at[pl.ds(off, B)], dstv[p], semi[p])
        pltpu.async_copy(rel_hbm.at[pl.ds(off, B)], relv[p], semi[p])

    def wait_idx(p):
        pltpu.make_async_copy(src_hbm.at[pl.ds(0, B)], srcv[p], semi[p]).wait()
        pltpu.make_async_copy(src_hbm.at[pl.ds(0, B)], dstv[p], semi[p]).wait()
        pltpu.make_async_copy(src_hbm.at[pl.ds(0, B)], relv[p], semi[p]).wait()

    def compute_idx(p, q):
        for j in range(B // LW):
            sl = pl.ds(j * LW, LW)
            s = srcv[p][sl]
            d = dstv[p][sl]
            r = relv[p][sl]
            gv[p][sl] = r * N + s
            sv[p][sl] = d * R + r
            dv[q][sl] = d

    def issue_gather(p):
        pltpu.async_copy(y_hbm.at[gv[p]], rows[p], semg[p])
        pltpu.async_copy(recip_hbm.at[sv[p]], scl[p], sems[p])

    def wait_gather(p):
        pltpu.make_async_copy(y_hbm.at[gv[p]], rows[p], semg[p]).wait()
        pltpu.make_async_copy(recip_hbm.at[sv[p]], scl[p], sems[p]).wait()

    def wait_scatter(p, q):
        pltpu.make_async_copy(sbuf[p], acc.at[dv[q]], semw[p]).wait()

    def scale_and_scatter(p, q):
        def edge(e, carry2):
            svec = scl[p][e]
            for j in range(D // LW):
                sl = pl.ds(j * LW, LW)
                sbuf[p][e, sl] = rows[p][e, sl] * svec
            return carry2
        lax.fori_loop(0, B, edge, 0)
        pltpu.async_copy(sbuf[p], acc.at[dv[q]], semw[p], add=True)

    issue_idx(0, 0)
    issue_idx(1, 1)
    wait_idx(0)
    compute_idx(0, 0)
    issue_gather(0)
    wait_idx(1)
    compute_idx(1, 1)
    issue_gather(1)
    issue_idx(2, 0)
    issue_idx(3, 1)

    _zero_vmem_2d(zv, _ZVA, D)
    for k in range(RPS // _ZVA):
        pltpu.sync_copy(zv, acc.at[pl.ds(sid * RPS + k * _ZVA, _ZVA)])
    plsc.subcore_barrier()

    def body4(i, carry):
        for k in range(4):
            c = 4 * i + k
            p = k % 2
            wait_gather(p)

            @pl.when(c >= 2)
            def _():
                wait_scatter(p, (k + 2) % 4)
            scale_and_scatter(p, k)

            @pl.when(c + 2 < NCHUNK)
            def _():
                wait_idx(p)
                compute_idx(p, (k + 2) % 4)
                issue_gather(p)

            @pl.when(c + 4 < NCHUNK)
            def _():
                issue_idx(c + 4, p)
        return carry
    lax.fori_loop(0, NCHUNK // 4, body4, 0)

    wait_gather(0)
    wait_scatter(0, 2)
    scale_and_scatter(0, 0)
    wait_scatter(1, 3)
    wait_scatter(0, 0)

    plsc.subcore_barrier()
    for k in range(RPS // _ZVA):
        s = sid * RPS + k * _ZVA
        pltpu.sync_copy(acc.at[pl.ds(s, _ZVA)], out_hbm.at[cid, pl.ds(s, _ZVA)])


_BN = 2000


def _recip_body(degp_ref, recip_ref):
    d = degp_ref[0] + degp_ref[1]
    recip_ref[...] = 1.0 / jnp.maximum(d, 1.0)


def _recip_tc(deg_parts):
    nb = 8
    blk = SEGN // nb
    return pl.pallas_call(
        _recip_body,
        grid=(nb,),
        in_specs=[pl.BlockSpec((NC, blk, LW), lambda i: (0, i, 0))],
        out_specs=pl.BlockSpec((blk, LW), lambda i: (i, 0)),
        out_shape=jax.ShapeDtypeStruct((SEGN, LW), jnp.float32),
    )(deg_parts)


def _transform_body(x_ref, w_ref, y_ref):
    y_ref[...] = jnp.dot(x_ref[...], w_ref[0], preferred_element_type=jnp.float32)


def _transform_tc(x, W):
    nbx = N // _BN
    return pl.pallas_call(
        _transform_body,
        grid=(R, nbx),
        in_specs=[
            pl.BlockSpec((_BN, D), lambda r, i: (i, 0)),
            pl.BlockSpec((1, D, D), lambda r, i: (r, 0, 0)),
        ],
        out_specs=pl.BlockSpec((_BN, D), lambda r, i: (r * nbx + i, 0)),
        out_shape=jax.ShapeDtypeStruct((R * N, D), jnp.float32),
    )(x, W)


def _combine_body(relu, parts_ref, x_ref, root_ref, b_ref, out_ref):
    acc = parts_ref[0] + parts_ref[1]
    acc = acc + jnp.dot(x_ref[...], root_ref[...], preferred_element_type=jnp.float32)
    acc = acc + b_ref[...]
    if relu:
        acc = jnp.maximum(acc, 0.0)
    out_ref[...] = acc


def _combine_tc(parts, x, root, b, relu):
    nbx = N // _BN
    return pl.pallas_call(
        functools.partial(_combine_body, relu),
        grid=(nbx,),
        in_specs=[
            pl.BlockSpec((NC, _BN, D), lambda i: (0, i, 0)),
            pl.BlockSpec((_BN, D), lambda i: (i, 0)),
            pl.BlockSpec((D, D), lambda i: (0, 0)),
            pl.BlockSpec((1, D), lambda i: (0, 0)),
        ],
        out_specs=pl.BlockSpec((_BN, D), lambda i: (i, 0)),
        out_shape=jax.ShapeDtypeStruct((N, D), jnp.float32),
    )(parts, x, root, b)


def _layer(x, W, root, b, src, dst, rel, recip, relu):
    y = _transform_tc(x, W)
    parts = _agg_kernel(y, recip, src, dst, rel)
    return _combine_tc(parts, x, root, b.reshape(1, D), relu)


def kernel(x, edge_index, edge_type, W1, root1, b1, W2, root2, b2):
    src = edge_index[0].astype(jnp.int32)
    dst = edge_index[1].astype(jnp.int32)
    rel = edge_type.astype(jnp.int32)
    deg_parts = _deg_kernel(dst, rel)
    recip = _recip_tc(deg_parts)
    h = _layer(x, W1, root1, b1, src, dst, rel, recip, relu=True)
    return _layer(h, W2, root2, b2, src, dst, rel, recip, relu=False)


# speedup vs baseline: 7.9968x; 1.0118x over previous
import functools

import jax
import jax.numpy as jnp
from jax import lax
from jax.experimental import pallas as pl
from jax.experimental.pallas import tpu as pltpu
from jax.experimental.pallas import tpu_sc as plsc

N = 10000
R = 8
E = 320000
D = 128
SEGN = N * R

NC = 2
NS = 16
NW = NC * NS
EPW = E // NW
B = 80
NCHUNK = EPW // B
RPS = N // NS
ZR = 125
SEGPS = SEGN // NS
SEGZ = 1000
LW = 16

_MESH = plsc.VectorSubcoreMesh(core_axis_name="c", subcore_axis_name="s")
_SC_PARAMS = pltpu.CompilerParams(use_tc_tiling_on_sc=False)


def _zero_vmem_2d(ref, nrows, ncols):
    def row(i, carry):
        for j in range(ncols // LW):
            ref[i, pl.ds(j * LW, LW)] = jnp.zeros((LW,), jnp.float32)
        return carry
    lax.fori_loop(0, nrows, row, 0)


_DEG_LAG = 4


@functools.partial(
    pl.kernel,
    out_type=jax.ShapeDtypeStruct((NC, SEGN, LW), jnp.float32),
    mesh=_MESH,
    scratch_types=[
        pltpu.VMEM((EPW,), jnp.int32),
        pltpu.VMEM((EPW,), jnp.int32),
        pltpu.VMEM((NCHUNK, B), jnp.int32),
        pltpu.VMEM((B, LW), jnp.float32),
        pltpu.VMEM((SEGZ, LW), jnp.float32),
        pltpu.VMEM_SHARED((SEGN, LW), jnp.float32),
        pltpu.SemaphoreType.DMA,
    ],
    compiler_params=_SC_PARAMS,
)
def _deg_kernel(dst_hbm, rel_hbm, out_hbm, dstw, relw, segw, onesv, zv, acc, sem):
    cid = lax.axis_index("c")
    sid = lax.axis_index("s")
    wid = cid * NS + sid
    base = wid * EPW

    pltpu.sync_copy(dst_hbm.at[pl.ds(base, EPW)], dstw)
    pltpu.sync_copy(rel_hbm.at[pl.ds(base, EPW)], relw)

    def seg_chunk(c, carry):
        off = c * B
        for j in range(B // LW):
            sl = pl.ds(j * LW, LW)
            s2 = pl.ds(off + j * LW, LW)
            segw[c, sl] = dstw[s2] * R + relw[s2]
        return carry
    lax.fori_loop(0, NCHUNK, seg_chunk, 0)

    def orow(i, carry):
        onesv[i, pl.ds(0, LW)] = jnp.ones((LW,), jnp.float32)
        return carry
    lax.fori_loop(0, B, orow, 0)

    _zero_vmem_2d(zv, SEGZ, LW)
    for k in range(SEGPS // SEGZ):
        pltpu.sync_copy(zv, acc.at[pl.ds(sid * SEGPS + k * SEGZ, SEGZ)])
    plsc.subcore_barrier()

    def chunk(c, carry):
        pltpu.async_copy(onesv, acc.at[segw.at[c]], sem, add=True)

        @pl.when(c >= _DEG_LAG)
        def _():
            pltpu.make_async_copy(onesv, acc.at[segw.at[0]], sem).wait()
        return carry
    lax.fori_loop(0, NCHUNK, chunk, 0)
    for _ in range(_DEG_LAG):
        pltpu.make_async_copy(onesv, acc.at[segw.at[0]], sem).wait()

    plsc.subcore_barrier()
    for k in range(SEGPS // SEGZ):
        s = sid * SEGPS + k * SEGZ
        pltpu.sync_copy(acc.at[pl.ds(s, SEGZ)], out_hbm.at[cid, pl.ds(s, SEGZ)])


@functools.partial(
    pl.kernel,
    out_type=jax.ShapeDtypeStruct((NC, N, D), jnp.float32),
    mesh=_MESH,
    scratch_types=[
        [pltpu.VMEM((B,), jnp.int32)] * 2,
        [pltpu.VMEM((B,), jnp.int32)] * 2,
        [pltpu.VMEM((B,), jnp.int32)] * 2,
        [pltpu.VMEM((B,), jnp.int32)] * 2,
        [pltpu.VMEM((B,), jnp.int32)] * 2,
        [pltpu.VMEM((B,), jnp.int32)] * 4,
        [pltpu.VMEM((B, D), jnp.float32)] * 2,
        [pltpu.VMEM((B, D), jnp.float32)] * 2,
        [pltpu.VMEM((B, LW), jnp.float32)] * 2,
        pltpu.VMEM_SHARED((N, D), jnp.float32),
        [pltpu.SemaphoreType.DMA] * 2,
        [pltpu.SemaphoreType.DMA] * 2,
        [pltpu.SemaphoreType.DMA] * 2,
        [pltpu.SemaphoreType.DMA] * 2,
        pltpu.SemaphoreType.DMA,
    ],
    compiler_params=_SC_PARAMS,
)
def _agg_kernel(y_hbm, recip_hbm, src_hbm, dst_hbm, rel_hbm, zeros_hbm, out_hbm,
                srcv, dstv, relv, gv, sv, dv, rows, sbuf, scl, acc,
                semi, semg, sems, semw, semz):
    cid = lax.axis_index("c")
    sid = lax.axis_index("s")
    wid = cid * NS + sid
    base = wid * EPW

    def issue_idx(c, p):
        off = base + c * B
        pltpu.async_copy(src_hbm.at[pl.ds(off, B)], srcv[p], semi[p])
        pltpu.async_copy(dst_hbm.at[pl.ds(off, B)], dstv[p], semi[p])
        pltpu.async_copy(rel_hbm.at[pl.ds(off, B)], relv[p], semi[p])

    def wait_idx(p):
        pltpu.make_async_copy(src_hbm.at[pl.ds(0, B)], srcv[p], semi[p]).wait()
        pltpu.make_async_copy(src_hbm.at[pl.ds(0, B)], dstv[p], semi[p]).wait()
        pltpu.make_async_copy(src_hbm.at[pl.ds(0, B)], relv[p], semi[p]).wait()

    def compute_idx(p, q):
        for j in range(B // LW):
            sl = pl.ds(j * LW, LW)
            s = srcv[p][sl]
            d = dstv[p][sl]
            r = relv[p][sl]
            gv[p][sl] = r * N + s
            sv[p][sl] = d * R + r
            dv[q][sl] = d

    def issue_gather(p):
        pltpu.async_copy(y_hbm.at[gv[p]], rows[p], semg[p])
        pltpu.async_copy(recip_hbm.at[sv[p]], scl[p], sems[p])

    def wait_gather(p):
        pltpu.make_async_copy(y_hbm.at[gv[p]], rows[p], semg[p]).wait()
        pltpu.make_async_copy(recip_hbm.at[sv[p]], scl[p], sems[p]).wait()

    def wait_scatter(p, q):
        pltpu.make_async_copy(sbuf[p], acc.at[dv[q]], semw[p]).wait()

    def scale_and_scatter(p, q):
        def edge(e2, carry2):
            for u in range(2):
                e = e2 * 2 + u
                svec = scl[p][e]
                for j in range(D // LW):
                    sl = pl.ds(j * LW, LW)
                    sbuf[p][e, sl] = rows[p][e, sl] * svec
            return carry2
        lax.fori_loop(0, B // 2, edge, 0)
        pltpu.async_copy(sbuf[p], acc.at[dv[q]], semw[p], add=True)

    issue_idx(0, 0)
    issue_idx(1, 1)
    zcp = pltpu.async_copy(
        zeros_hbm.at[pl.ds(sid * RPS, RPS)], acc.at[pl.ds(sid * RPS, RPS)], semz)
    wait_idx(0)
    compute_idx(0, 0)
    issue_gather(0)
    wait_idx(1)
    compute_idx(1, 1)
    issue_gather(1)
    issue_idx(2, 0)
    issue_idx(3, 1)
    zcp.wait()
    plsc.subcore_barrier()

    def body4(i, carry):
        for k in range(4):
            c = 4 * i + k
            p = k % 2
            wait_gather(p)

            @pl.when(c >= 2)
            def _():
                wait_scatter(p, (k + 2) % 4)
            scale_and_scatter(p, k)

            @pl.when(c + 2 < NCHUNK)
            def _():
                wait_idx(p)
                compute_idx(p, (k + 2) % 4)
                issue_gather(p)

            @pl.when(c + 4 < NCHUNK)
            def _():
                issue_idx(c + 4, p)
        return carry
    lax.fori_loop(0, NCHUNK // 4, body4, 0)

    wait_gather(0)
    wait_scatter(0, 2)
    scale_and_scatter(0, 0)
    wait_scatter(1, 3)
    wait_scatter(0, 0)

    plsc.subcore_barrier()
    s = sid * RPS
    pltpu.sync_copy(acc.at[pl.ds(s, RPS)], out_hbm.at[cid, pl.ds(s, RPS)])


_BN = 2000
_NBX = N // _BN
_SEGB = SEGN // (_NBX * R)


def _transform_recip_body(x_ref, w_ref, degp_ref, y_ref, recip_ref):
    y_ref[...] = jnp.dot(x_ref[...], w_ref[0], preferred_element_type=jnp.float32)
    d = degp_ref[0] + degp_ref[1]
    recip_ref[...] = 1.0 / jnp.maximum(d, 1.0)


def _transform_recip_tc(x, W, deg_parts):
    return pl.pallas_call(
        _transform_recip_body,
        grid=(_NBX, R),
        in_specs=[
            pl.BlockSpec((_BN, D), lambda i, r: (i, 0)),
            pl.BlockSpec((1, D, D), lambda i, r: (r, 0, 0)),
            pl.BlockSpec((NC, _SEGB, LW), lambda i, r: (0, r * _NBX + i, 0)),
        ],
        out_specs=[
            pl.BlockSpec((_BN, D), lambda i, r: (r * _NBX + i, 0)),
            pl.BlockSpec((_SEGB, LW), lambda i, r: (r * _NBX + i, 0)),
        ],
        out_shape=[
            jax.ShapeDtypeStruct((R * N, D), jnp.float32),
            jax.ShapeDtypeStruct((SEGN, LW), jnp.float32),
        ],
    )(x, W, deg_parts)


def _combine_transform_body(parts_ref, x_ref, root_ref, b_ref, w_ref,
                            y_ref, h_ref):
    h = parts_ref[0] + parts_ref[1]
    h = h + jnp.dot(x_ref[...], root_ref[...], preferred_element_type=jnp.float32)
    h = jnp.maximum(h + b_ref[...], 0.0)
    h_ref[...] = h
    y_ref[...] = jnp.dot(h, w_ref[0], preferred_element_type=jnp.float32)


def _combine_transform_tc(parts, x, root, b, W2):
    return pl.pallas_call(
        _combine_transform_body,
        grid=(_NBX, R),
        in_specs=[
            pl.BlockSpec((NC, _BN, D), lambda i, r: (0, i, 0)),
            pl.BlockSpec((_BN, D), lambda i, r: (i, 0)),
            pl.BlockSpec((D, D), lambda i, r: (0, 0)),
            pl.BlockSpec((1, D), lambda i, r: (0, 0)),
            pl.BlockSpec((1, D, D), lambda i, r: (r, 0, 0)),
        ],
        out_specs=[
            pl.BlockSpec((_BN, D), lambda i, r: (r * _NBX + i, 0)),
            pl.BlockSpec((_BN, D), lambda i, r: (i, 0)),
        ],
        out_shape=[
            jax.ShapeDtypeStruct((R * N, D), jnp.float32),
            jax.ShapeDtypeStruct((N, D), jnp.float32),
        ],
    )(parts, x, root, b, W2)


def _combine_body(parts_ref, x_ref, root_ref, b_ref, out_ref):
    acc = parts_ref[0] + parts_ref[1]
    acc = acc + jnp.dot(x_ref[...], root_ref[...], preferred_element_type=jnp.float32)
    out_ref[...] = acc + b_ref[...]


def _combine_tc(parts, x, root, b):
    return pl.pallas_call(
        _combine_body,
        grid=(_NBX,),
        in_specs=[
            pl.BlockSpec((NC, _BN, D), lambda i: (0, i, 0)),
            pl.BlockSpec((_BN, D), lambda i: (i, 0)),
            pl.BlockSpec((D, D), lambda i: (0, 0)),
            pl.BlockSpec((1, D), lambda i: (0, 0)),
        ],
        out_specs=pl.BlockSpec((_BN, D), lambda i: (i, 0)),
        out_shape=jax.ShapeDtypeStruct((N, D), jnp.float32),
    )(parts, x, root, b)


def kernel(x, edge_index, edge_type, W1, root1, b1, W2, root2, b2):
    src = edge_index[0].astype(jnp.int32)
    dst = edge_index[1].astype(jnp.int32)
    rel = edge_type.astype(jnp.int32)
    zeros = jnp.zeros((N, D), jnp.float32)
    deg_parts = _deg_kernel(dst, rel)
    y1, recip = _transform_recip_tc(x, W1, deg_parts)
    parts1 = _agg_kernel(y1, recip, src, dst, rel, zeros)
    y2, h = _combine_transform_tc(parts1, x, root1, b1.reshape(1, D), W2)
    parts2 = _agg_kernel(y2, recip, src, dst, rel, zeros)
    return _combine_tc(parts2, h, root2, b2.reshape(1, D))


# speedup vs baseline: 10.0197x; 1.2530x over previous
import functools

import jax
import jax.numpy as jnp
from jax import lax
from jax.experimental import pallas as pl
from jax.experimental.pallas import tpu as pltpu
from jax.experimental.pallas import tpu_sc as plsc

N = 10000
R = 8
E = 320000
D = 128
SEGN = N * R

NC = 2
NS = 16
NW = NC * NS
EPW = E // NW
B = 80
NCHUNK = EPW // B
RPS = N // NS
ZR = 125
SEGPS = SEGN // NS
SEGZ = 1000
LW = 16

_MESH = plsc.VectorSubcoreMesh(core_axis_name="c", subcore_axis_name="s")
_SC_PARAMS = pltpu.CompilerParams(use_tc_tiling_on_sc=False)


def _zero_vmem_2d(ref, nrows, ncols):
    def row(i, carry):
        for j in range(ncols // LW):
            ref[i, pl.ds(j * LW, LW)] = jnp.zeros((LW,), jnp.float32)
        return carry
    lax.fori_loop(0, nrows, row, 0)


_DEG_LAG = 4


@functools.partial(
    pl.kernel,
    out_type=jax.ShapeDtypeStruct((NC, SEGN, LW), jnp.float32),
    mesh=_MESH,
    scratch_types=[
        pltpu.VMEM((EPW,), jnp.int32),
        pltpu.VMEM((EPW,), jnp.int32),
        pltpu.VMEM((NCHUNK, B), jnp.int32),
        pltpu.VMEM((B, LW), jnp.float32),
        pltpu.VMEM((SEGZ, LW), jnp.float32),
        pltpu.VMEM_SHARED((SEGN, LW), jnp.float32),
        pltpu.SemaphoreType.DMA,
    ],
    compiler_params=_SC_PARAMS,
)
def _deg_kernel(ei_hbm, et_hbm, out_hbm, dstw, relw, segw, onesv, zv, acc, sem):
    cid = lax.axis_index("c")
    sid = lax.axis_index("s")
    wid = cid * NS + sid
    base = wid * EPW

    pltpu.sync_copy(ei_hbm.at[pl.ds(E + base, EPW)], dstw)
    pltpu.sync_copy(et_hbm.at[pl.ds(base, EPW)], relw)

    def seg_chunk(c, carry):
        off = c * B
        for j in range(B // LW):
            sl = pl.ds(j * LW, LW)
            s2 = pl.ds(off + j * LW, LW)
            segw[c, sl] = dstw[s2] * R + relw[s2]
        return carry
    lax.fori_loop(0, NCHUNK, seg_chunk, 0)

    def orow(i, carry):
        onesv[i, pl.ds(0, LW)] = jnp.ones((LW,), jnp.float32)
        return carry
    lax.fori_loop(0, B, orow, 0)

    _zero_vmem_2d(zv, SEGZ, LW)
    for k in range(SEGPS // SEGZ):
        pltpu.sync_copy(zv, acc.at[pl.ds(sid * SEGPS + k * SEGZ, SEGZ)])
    plsc.subcore_barrier()

    def chunk(c, carry):
        pltpu.async_copy(onesv, acc.at[segw.at[c]], sem, add=True)

        @pl.when(c >= _DEG_LAG)
        def _():
            pltpu.make_async_copy(onesv, acc.at[segw.at[0]], sem).wait()
        return carry
    lax.fori_loop(0, NCHUNK, chunk, 0)
    for _ in range(_DEG_LAG):
        pltpu.make_async_copy(onesv, acc.at[segw.at[0]], sem).wait()

    plsc.subcore_barrier()
    for k in range(SEGPS // SEGZ):
        s = sid * SEGPS + k * SEGZ
        pltpu.sync_copy(acc.at[pl.ds(s, SEGZ)], out_hbm.at[cid, pl.ds(s, SEGZ)])


@functools.partial(
    pl.kernel,
    out_type=jax.ShapeDtypeStruct((NC, N, D), jnp.float32),
    mesh=_MESH,
    scratch_types=[
        [pltpu.VMEM((B,), jnp.int32)] * 2,
        [pltpu.VMEM((B,), jnp.int32)] * 2,
        [pltpu.VMEM((B,), jnp.int32)] * 2,
        [pltpu.VMEM((B,), jnp.int32)] * 2,
        [pltpu.VMEM((B,), jnp.int32)] * 2,
        [pltpu.VMEM((B,), jnp.int32)] * 4,
        [pltpu.VMEM((B, D), jnp.float32)] * 2,
        [pltpu.VMEM((B, D), jnp.float32)] * 2,
        [pltpu.VMEM((B, LW), jnp.float32)] * 2,
        pltpu.VMEM_SHARED((N, D), jnp.float32),
        [pltpu.SemaphoreType.DMA] * 2,
        [pltpu.SemaphoreType.DMA] * 2,
        [pltpu.SemaphoreType.DMA] * 2,
        [pltpu.SemaphoreType.DMA] * 2,
        pltpu.SemaphoreType.DMA,
    ],
    compiler_params=_SC_PARAMS,
)
def _agg_kernel(y_hbm, recip_hbm, ei_hbm, et_hbm, zeros_hbm, out_hbm,
                srcv, dstv, relv, gv, sv, dv, rows, sbuf, scl, acc,
                semi, semg, sems, semw, semz):
    cid = lax.axis_index("c")
    sid = lax.axis_index("s")
    wid = cid * NS + sid
    base = wid * EPW

    def issue_idx(c, p):
        off = base + c * B
        pltpu.async_copy(ei_hbm.at[pl.ds(off, B)], srcv[p], semi[p])
        pltpu.async_copy(ei_hbm.at[pl.ds(E + off, B)], dstv[p], semi[p])
        pltpu.async_copy(et_hbm.at[pl.ds(off, B)], relv[p], semi[p])

    def wait_idx(p):
        pltpu.make_async_copy(et_hbm.at[pl.ds(0, B)], srcv[p], semi[p]).wait()
        pltpu.make_async_copy(et_hbm.at[pl.ds(0, B)], dstv[p], semi[p]).wait()
        pltpu.make_async_copy(et_hbm.at[pl.ds(0, B)], relv[p], semi[p]).wait()

    def compute_idx(p, q):
        for j in range(B // LW):
            sl = pl.ds(j * LW, LW)
            s = srcv[p][sl]
            d = dstv[p][sl]
            r = relv[p][sl]
            gv[p][sl] = r * N + s
            sv[p][sl] = d * R + r
            dv[q][sl] = d

    def issue_gather(p):
        pltpu.async_copy(y_hbm.at[gv[p]], rows[p], semg[p])
        pltpu.async_copy(recip_hbm.at[sv[p]], scl[p], sems[p])

    def wait_gather(p):
        pltpu.make_async_copy(y_hbm.at[gv[p]], rows[p], semg[p]).wait()
        pltpu.make_async_copy(recip_hbm.at[sv[p]], scl[p], sems[p]).wait()

    def wait_scatter(p, q):
        pltpu.make_async_copy(sbuf[p], acc.at[dv[q]], semw[p]).wait()

    def scale_and_scatter(p, q):
        def edge(e2, carry2):
            for u in range(2):
                e = e2 * 2 + u
                svec = scl[p][e]
                for j in range(D // LW):
                    sl = pl.ds(j * LW, LW)
                    sbuf[p][e, sl] = rows[p][e, sl] * svec
            return carry2
        lax.fori_loop(0, B // 2, edge, 0)
        pltpu.async_copy(sbuf[p], acc.at[dv[q]], semw[p], add=True)

    issue_idx(0, 0)
    issue_idx(1, 1)
    zcp = pltpu.async_copy(
        zeros_hbm.at[pl.ds(sid * RPS, RPS)], acc.at[pl.ds(sid * RPS, RPS)], semz)
    wait_idx(0)
    compute_idx(0, 0)
    issue_gather(0)
    wait_idx(1)
    compute_idx(1, 1)
    issue_gather(1)
    issue_idx(2, 0)
    issue_idx(3, 1)
    zcp.wait()
    plsc.subcore_barrier()

    def body4(i, carry):
        for k in range(4):
            c = 4 * i + k
            p = k % 2
            wait_gather(p)

            @pl.when(c >= 2)
            def _():
                wait_scatter(p, (k + 2) % 4)
            scale_and_scatter(p, k)

            @pl.when(c + 2 < NCHUNK)
            def _():
                wait_idx(p)
                compute_idx(p, (k + 2) % 4)
                issue_gather(p)

            @pl.when(c + 4 < NCHUNK)
            def _():
                issue_idx(c + 4, p)
        return carry
    lax.fori_loop(0, NCHUNK // 4, body4, 0)

    wait_gather(0)
    wait_scatter(0, 2)
    scale_and_scatter(0, 0)
    wait_scatter(1, 3)
    wait_scatter(0, 0)

    plsc.subcore_barrier()
    s = sid * RPS
    pltpu.sync_copy(acc.at[pl.ds(s, RPS)], out_hbm.at[cid, pl.ds(s, RPS)])


_RRW = SEGN // NW


@functools.partial(
    pl.kernel,
    out_type=jax.ShapeDtypeStruct((SEGN, LW), jnp.float32),
    mesh=_MESH,
    scratch_types=[
        pltpu.VMEM((_RRW, LW), jnp.float32),
        pltpu.VMEM((_RRW, LW), jnp.float32),
    ],
    compiler_params=_SC_PARAMS,
)
def _recip_kernel(degp_hbm, recip_hbm, d0, d1):
    cid = lax.axis_index("c")
    sid = lax.axis_index("s")
    wid = cid * NS + sid
    s = wid * _RRW
    pltpu.sync_copy(degp_hbm.at[0, pl.ds(s, _RRW)], d0)
    pltpu.sync_copy(degp_hbm.at[1, pl.ds(s, _RRW)], d1)

    def row(r, carry):
        d = d0[r] + d1[r]
        d0[r] = 1.0 / jnp.maximum(d, 1.0)
        return carry
    lax.fori_loop(0, _RRW, row, 0)
    pltpu.sync_copy(d0, recip_hbm.at[pl.ds(s, _RRW)])


_BN = 2000
_NBX = N // _BN


def _transform_body(x_ref, w_ref, y_ref):
    y_ref[...] = jnp.dot(x_ref[...], w_ref[0], preferred_element_type=jnp.float32)


def _transform_tc(x, W):
    return pl.pallas_call(
        _transform_body,
        grid=(_NBX, R),
        in_specs=[
            pl.BlockSpec((_BN, D), lambda i, r: (i, 0)),
            pl.BlockSpec((1, D, D), lambda i, r: (r, 0, 0)),
        ],
        out_specs=pl.BlockSpec((_BN, D), lambda i, r: (r * _NBX + i, 0)),
        out_shape=jax.ShapeDtypeStruct((R * N, D), jnp.float32),
    )(x, W)


def _combine_transform_body(parts_ref, x_ref, root_ref, b_ref, w_ref,
                            y_ref, h_ref):
    h = parts_ref[0] + parts_ref[1]
    h = h + jnp.dot(x_ref[...], root_ref[...], preferred_element_type=jnp.float32)
    h = jnp.maximum(h + b_ref[...], 0.0)
    h_ref[...] = h
    y_ref[...] = jnp.dot(h, w_ref[0], preferred_element_type=jnp.float32)


def _combine_transform_tc(parts, x, root, b, W2):
    return pl.pallas_call(
        _combine_transform_body,
        grid=(_NBX, R),
        in_specs=[
            pl.BlockSpec((NC, _BN, D), lambda i, r: (0, i, 0)),
            pl.BlockSpec((_BN, D), lambda i, r: (i, 0)),
            pl.BlockSpec((D, D), lambda i, r: (0, 0)),
            pl.BlockSpec((1, D), lambda i, r: (0, 0)),
            pl.BlockSpec((1, D, D), lambda i, r: (r, 0, 0)),
        ],
        out_specs=[
            pl.BlockSpec((_BN, D), lambda i, r: (r * _NBX + i, 0)),
            pl.BlockSpec((_BN, D), lambda i, r: (i, 0)),
        ],
        out_shape=[
            jax.ShapeDtypeStruct((R * N, D), jnp.float32),
            jax.ShapeDtypeStruct((N, D), jnp.float32),
        ],
    )(parts, x, root, b, W2)


def _combine_body(parts_ref, x_ref, root_ref, b_ref, out_ref):
    acc = parts_ref[0] + parts_ref[1]
    acc = acc + jnp.dot(x_ref[...], root_ref[...], preferred_element_type=jnp.float32)
    out_ref[...] = acc + b_ref[...]


def _combine_tc(parts, x, root, b):
    return pl.pallas_call(
        _combine_body,
        grid=(_NBX,),
        in_specs=[
            pl.BlockSpec((NC, _BN, D), lambda i: (0, i, 0)),
            pl.BlockSpec((_BN, D), lambda i: (i, 0)),
            pl.BlockSpec((D, D), lambda i: (0, 0)),
            pl.BlockSpec((1, D), lambda i: (0, 0)),
        ],
        out_specs=pl.BlockSpec((_BN, D), lambda i: (i, 0)),
        out_shape=jax.ShapeDtypeStruct((N, D), jnp.float32),
    )(parts, x, root, b)


def kernel(x, edge_index, edge_type, W1, root1, b1, W2, root2, b2):
    ei = edge_index.astype(jnp.int32).reshape(2 * E)
    et = edge_type.astype(jnp.int32)
    zeros = jnp.zeros((N, D), jnp.float32)
    deg_parts = _deg_kernel(ei, et)
    recip = _recip_kernel(deg_parts)
    y1 = _transform_tc(x, W1)
    parts1 = _agg_kernel(y1, recip, ei, et, zeros)
    y2, h = _combine_transform_tc(parts1, x, root1, b1.reshape(1, D), W2)
    parts2 = _agg_kernel(y2, recip, ei, et, zeros)
    return _combine_tc(parts2, h, root2, b2.reshape(1, D))
